# Initial kernel scaffold; baseline (speedup 1.0000x reference)
import jax
import jax.numpy as jnp
from jax.experimental import pallas as pl


def kernel(features, rois):
    raise NotImplementedError("write your pallas kernel here")


# speedup vs baseline: 1.2387x; 1.2387x over previous
import functools

import jax
import jax.numpy as jnp
from jax import lax
from jax.experimental import pallas as pl
from jax.experimental.pallas import tpu as pltpu
from jax.experimental.pallas import tpu_sc as plsc

P = 7
G = 2
S = P * G
NCH = 192
CV = NCH // 16


def _roi_align_sc(table, rois16, K, H, W):
    info = plsc.get_sparse_core_info()
    NC, NS = info.num_cores, info.num_subcores
    NW = NC * NS
    RPW = K // NW
    assert RPW * NW == K

    mesh = plsc.VectorSubcoreMesh(core_axis_name="c", subcore_axis_name="s")
    f32, i32 = jnp.float32, jnp.int32

    def body(table_ref, rois_ref, out_ref, roi_v, ybase_s, hy_s, ly_s, w_ref,
             idx0, idx1, idx2, idx3, idx4, idx5, idx6,
             rows0, rows1, acc, sem0, sem1):
        idx_refs = [idx0, idx1, idx2, idx3, idx4, idx5, idx6]
        wid = lax.axis_index("s") * NC + lax.axis_index("c")
        lane = lax.iota(i32, 16)
        lanef = lane.astype(f32)
        active = lane < S

        def bc(ref, i):
            return plsc.load_gather(ref, [jnp.full((16,), i, i32)])

        def run_roi(kk, carry):
            k = wid * RPW + kk
            pltpu.sync_copy(rois_ref.at[k], roi_v)
            bf = bc(roi_v, 0)
            x1 = bc(roi_v, 1)
            y1 = bc(roi_v, 2)
            x2 = bc(roi_v, 3)
            y2 = bc(roi_v, 4)
            bw = jnp.maximum(x2 - x1, 1.0) * (1.0 / P)
            bh = jnp.maximum(y2 - y1, 1.0) * (1.0 / P)
            off = lanef * (1.0 / G) + (0.5 / G)
            xs = x1 + off * bw
            ys = y1 + off * bh
            cx = jnp.maximum(xs, 0.0)
            xli = cx.astype(i32)
            xcond = xli >= (W - 1)
            xli = jnp.where(xcond, W - 1, xli)
            lx = jnp.where(xcond, 0.0, cx - xli.astype(f32))
            hx = 1.0 - lx
            xli = jnp.where(active, xli, 0)
            lx = jnp.where(active, lx, 0.0)
            hx = jnp.where(active, hx, 0.0)
            cy = jnp.maximum(ys, 0.0)
            yli = cy.astype(i32)
            ycond = yli >= (H - 1)
            yli = jnp.where(ycond, H - 1, yli)
            ly = jnp.where(ycond, 0.0, cy - yli.astype(f32))
            hy = 1.0 - ly
            ybase = bf.astype(i32) * (H * W) + yli * W
            ybase_s[...] = ybase
            hy_s[...] = hy * 0.25
            ly_s[...] = ly * 0.25

            for iy in range(S):
                yb = bc(ybase_s, iy)
                hyb = bc(hy_s, iy)
                lyb = bc(ly_s, iy)
                ill = yb + xli
                idxj = idx_refs[iy // 2]
                base = (iy % 2) * 64
                idxj[pl.ds(base, 16)] = ill
                idxj[pl.ds(base + 16, 16)] = ill + 1
                idxj[pl.ds(base + 32, 16)] = ill + W
                idxj[pl.ds(base + 48, 16)] = ill + (W + 1)
                w_ref[pl.ds(iy * 64, 16)] = hyb * hx
                w_ref[pl.ds(iy * 64 + 16, 16)] = hyb * lx
                w_ref[pl.ds(iy * 64 + 32, 16)] = lyb * hx
                w_ref[pl.ds(iy * 64 + 48, 16)] = lyb * lx

            def zero(i, c):
                acc[pl.ds(i * 16, 16)] = jnp.zeros((16,), f32)
                return c
            lax.fori_loop(0, P * P * CV, zero, 0)

            bufs = [rows0, rows1]
            sems = [sem0, sem1]
            cps = {0: pltpu.async_copy(table_ref.at[idx_refs[0]], rows0, sem0)}
            for j in range(P):
                cps[j].wait()
                if j + 1 < P:
                    cps[j + 1] = pltpu.async_copy(
                        table_ref.at[idx_refs[j + 1]],
                        bufs[(j + 1) % 2], sems[(j + 1) % 2])
                rows = bufs[j % 2]
                for half in range(2):
                    iy = 2 * j + half
                    rb = half * 64

                    def point(ix, c, iy=iy, rb=rb, rows=rows, j=j):
                        wb = iy * 64 + ix
                        w0 = plsc.load_gather(w_ref, [jnp.full((16,), wb, i32)])
                        w1 = plsc.load_gather(w_ref, [jnp.full((16,), wb + 16, i32)])
                        w2 = plsc.load_gather(w_ref, [jnp.full((16,), wb + 32, i32)])
                        w3 = plsc.load_gather(w_ref, [jnp.full((16,), wb + 48, i32)])
                        accoff = (j * P + ix // 2) * NCH
                        r0 = rb + ix
                        for cv in range(CV):
                            co = pl.ds(cv * 16, 16)
                            v = (w0 * rows[r0, co] + w1 * rows[r0 + 16, co]
                                 + w2 * rows[r0 + 32, co] + w3 * rows[r0 + 48, co])
                            plsc.addupdate(acc.at[pl.ds(accoff + cv * 16, 16)], v)
                        return c
                    lax.fori_loop(0, S, point, 0)
            pltpu.sync_copy(acc, out_ref.at[k])
            return carry

        lax.fori_loop(0, RPW, run_roi, 0)

    fn = pl.kernel(
        body,
        out_type=jax.ShapeDtypeStruct((K, P * P * NCH), jnp.float32),
        mesh=mesh,
        compiler_params=pltpu.CompilerParams(
            needs_layout_passes=False, use_tc_tiling_on_sc=False),
        scratch_types=(
            [pltpu.VMEM((16,), f32),
             pltpu.VMEM((16,), i32),
             pltpu.VMEM((16,), f32),
             pltpu.VMEM((16,), f32),
             pltpu.VMEM((S * 64,), f32)]
            + [pltpu.VMEM((128,), i32) for _ in range(P)]
            + [pltpu.VMEM((128, NCH), f32),
               pltpu.VMEM((128, NCH), f32),
               pltpu.VMEM((P * P * NCH,), f32),
               pltpu.SemaphoreType.DMA,
               pltpu.SemaphoreType.DMA]),
    )
    return fn(table, rois16)


@jax.jit
def kernel(features, rois):
    B, C, H, W = features.shape
    K = rois.shape[0]
    table = features.transpose(0, 2, 3, 1).reshape(B * H * W, C)
    table = jnp.concatenate(
        [table, jnp.zeros((W + 8, C), jnp.float32)], axis=0)
    rois16 = jnp.pad(rois, ((0, 0), (0, 11)))
    flat = _roi_align_sc(table, rois16, K, H, W)
    return flat.reshape(K, P * P, C).transpose(0, 2, 1).reshape(K, C, P, P)


# speedup vs baseline: 3.2528x; 2.6259x over previous
import jax
import jax.numpy as jnp
from jax import lax
from jax.experimental import pallas as pl
from jax.experimental.pallas import tpu as pltpu
from jax.experimental.pallas import tpu_sc as plsc

P = 7
G = 2
S = P * G
NCH = 192
CV = NCH // 16
CA = 128
CB = NCH - CA


def _nchw_to_tables(features):
    B, C, H, W = features.shape
    HT = 8
    N = B * (H // HT)

    def body(in_ref, a_ref, b_ref):
        g = pl.program_id(0)

        @pl.when(g < N)
        def _():
            for y in range(HT):
                a_ref[pl.ds(y * W, W), :] = in_ref[0, :CA, y, :].T
                b_ref[pl.ds(y * W, W), :CB] = in_ref[0, CA:, y, :].T

        @pl.when(g >= N)
        def _():
            a_ref[...] = jnp.zeros((HT * W, CA), jnp.float32)
            b_ref[...] = jnp.zeros((HT * W, CA), jnp.float32)

    return pl.pallas_call(
        body,
        grid=(N + 1,),
        in_specs=[pl.BlockSpec(
            (1, C, HT, W),
            lambda g: (jnp.minimum(g // (H // HT), B - 1),
                       0, jnp.minimum(g % (H // HT), H // HT - 1), 0))],
        out_specs=[pl.BlockSpec((HT * W, CA), lambda g: (g, 0)),
                   pl.BlockSpec((HT * W, CA), lambda g: (g, 0))],
        out_shape=[jax.ShapeDtypeStruct(((N + 1) * HT * W, CA), jnp.float32),
                   jax.ShapeDtypeStruct(((N + 1) * HT * W, CA), jnp.float32)],
    )(features)


def _roi_align_sc(table_a, table_b, rois16, K, H, W):
    info = plsc.get_sparse_core_info()
    NC, NS = info.num_cores, info.num_subcores
    NW = NC * NS
    RPW = K // NW
    assert RPW * NW == K

    mesh = plsc.VectorSubcoreMesh(core_axis_name="c", subcore_axis_name="s")
    f32, i32 = jnp.float32, jnp.int32

    def body(ta_ref, tb_ref, rois_ref, out_ref, roi_v, ybase_s, hy_s, ly_s,
             w_ref, idx0, idx1, idx2, idx3, idx4, idx5, idx6,
             rowsa0, rowsa1, rowsb0, rowsb1, acc, sem0, sem1):
        idx_refs = [idx0, idx1, idx2, idx3, idx4, idx5, idx6]
        wid = lax.axis_index("s") * NC + lax.axis_index("c")
        lane = lax.iota(i32, 16)
        lanef = lane.astype(f32)
        active = lane < S

        def bc(ref, i):
            return plsc.load_gather(ref, [jnp.full((16,), i, i32)])

        def run_roi(kk, carry):
            k = wid * RPW + kk
            pltpu.sync_copy(rois_ref.at[k], roi_v)
            bf = bc(roi_v, 0)
            x1 = bc(roi_v, 1)
            y1 = bc(roi_v, 2)
            x2 = bc(roi_v, 3)
            y2 = bc(roi_v, 4)
            bw = jnp.maximum(x2 - x1, 1.0) * (1.0 / P)
            bh = jnp.maximum(y2 - y1, 1.0) * (1.0 / P)
            off = lanef * (1.0 / G) + (0.5 / G)
            xs = x1 + off * bw
            ys = y1 + off * bh
            cx = jnp.maximum(xs, 0.0)
            xli = cx.astype(i32)
            xcond = xli >= (W - 1)
            xli = jnp.where(xcond, W - 1, xli)
            lx = jnp.where(xcond, 0.0, cx - xli.astype(f32))
            hx = 1.0 - lx
            xli = jnp.where(active, xli, 0)
            lx = jnp.where(active, lx, 0.0)
            hx = jnp.where(active, hx, 0.0)
            cy = jnp.maximum(ys, 0.0)
            yli = cy.astype(i32)
            ycond = yli >= (H - 1)
            yli = jnp.where(ycond, H - 1, yli)
            ly = jnp.where(ycond, 0.0, cy - yli.astype(f32))
            hy = 1.0 - ly
            ybase = bf.astype(i32) * (H * W) + yli * W
            ybase_s[...] = ybase
            hy_s[...] = hy * 0.25
            ly_s[...] = ly * 0.25

            for iy in range(S):
                yb = bc(ybase_s, iy)
                hyb = bc(hy_s, iy)
                lyb = bc(ly_s, iy)
                ill = yb + xli
                idxj = idx_refs[iy // 2]
                base = (iy % 2) * 64
                idxj[pl.ds(base, 16)] = ill
                idxj[pl.ds(base + 16, 16)] = ill + 1
                idxj[pl.ds(base + 32, 16)] = ill + W
                idxj[pl.ds(base + 48, 16)] = ill + (W + 1)
                w_ref[pl.ds(iy * 64, 16)] = hyb * hx
                w_ref[pl.ds(iy * 64 + 16, 16)] = hyb * lx
                w_ref[pl.ds(iy * 64 + 32, 16)] = lyb * hx
                w_ref[pl.ds(iy * 64 + 48, 16)] = lyb * lx

            def zero(i, c):
                acc[pl.ds(i * 16, 16)] = jnp.zeros((16,), f32)
                return c
            lax.fori_loop(0, P * P * CV, zero, 0)

            bufsa = [rowsa0, rowsa1]
            bufsb = [rowsb0, rowsb1]
            sems = [sem0, sem1]

            def fire(j):
                par = j % 2
                return (pltpu.async_copy(ta_ref.at[idx_refs[j]],
                                         bufsa[par], sems[par]),
                        pltpu.async_copy(tb_ref.at[idx_refs[j]],
                                         bufsb[par], sems[par]))

            cps = {0: fire(0)}
            for j in range(P):
                cps[j][0].wait()
                cps[j][1].wait()
                if j + 1 < P:
                    cps[j + 1] = fire(j + 1)
                ra = bufsa[j % 2]
                rb_buf = bufsb[j % 2]
                for half in range(2):
                    iy = 2 * j + half
                    rbase = half * 64

                    def point(ix, c, iy=iy, rbase=rbase, ra=ra,
                              rb_buf=rb_buf, j=j):
                        wb = iy * 64 + ix
                        w0 = plsc.load_gather(w_ref, [jnp.full((16,), wb, i32)])
                        w1 = plsc.load_gather(w_ref, [jnp.full((16,), wb + 16, i32)])
                        w2 = plsc.load_gather(w_ref, [jnp.full((16,), wb + 32, i32)])
                        w3 = plsc.load_gather(w_ref, [jnp.full((16,), wb + 48, i32)])
                        accoff = (j * P + ix // 2) * NCH
                        r0 = rbase + ix
                        for cv in range(CV):
                            if cv < CA // 16:
                                co = pl.ds(cv * 16, 16)
                                v = (w0 * ra[r0, co] + w1 * ra[r0 + 16, co]
                                     + w2 * ra[r0 + 32, co] + w3 * ra[r0 + 48, co])
                            else:
                                co = pl.ds((cv - CA // 16) * 16, 16)
                                v = (w0 * rb_buf[r0, co] + w1 * rb_buf[r0 + 16, co]
                                     + w2 * rb_buf[r0 + 32, co] + w3 * rb_buf[r0 + 48, co])
                            plsc.addupdate(acc.at[pl.ds(accoff + cv * 16, 16)], v)
                        return c
                    lax.fori_loop(0, S, point, 0)
            pltpu.sync_copy(acc, out_ref.at[k])
            return carry

        lax.fori_loop(0, RPW, run_roi, 0)

    fn = pl.kernel(
        body,
        out_type=jax.ShapeDtypeStruct((K, P * P * NCH), jnp.float32),
        mesh=mesh,
        compiler_params=pltpu.CompilerParams(
            needs_layout_passes=False, use_tc_tiling_on_sc=False),
        scratch_types=(
            [pltpu.VMEM((16,), f32),
             pltpu.VMEM((16,), i32),
             pltpu.VMEM((16,), f32),
             pltpu.VMEM((16,), f32)]
            + [pltpu.VMEM((S * 64,), f32)]
            + [pltpu.VMEM((128,), i32) for _ in range(P)]
            + [pltpu.VMEM((128, CA), f32),
               pltpu.VMEM((128, CA), f32),
               pltpu.VMEM((128, CA), f32),
               pltpu.VMEM((128, CA), f32),
               pltpu.VMEM((P * P * NCH,), f32),
               pltpu.SemaphoreType.DMA,
               pltpu.SemaphoreType.DMA]),
    )
    return fn(table_a, table_b, rois16)


@jax.jit
def kernel(features, rois):
    B, C, H, W = features.shape
    K = rois.shape[0]
    table_a, table_b = _nchw_to_tables(features)
    rois16 = jnp.pad(rois, ((0, 0), (0, 11)))
    flat = _roi_align_sc(table_a, table_b, rois16, K, H, W)
    return flat.reshape(K, P * P, C).transpose(0, 2, 1).reshape(K, C, P, P)


# speedup vs baseline: 3.9956x; 1.2283x over previous
import jax
import jax.numpy as jnp
from jax import lax
from jax.experimental import pallas as pl
from jax.experimental.pallas import tpu as pltpu
from jax.experimental.pallas import tpu_sc as plsc

P = 7
G = 2
S = P * G
NCH = 192
CV = NCH // 16
CA = 128
CB = NCH - CA


def _nchw_to_tables(features):
    B, C, H, W = features.shape
    HT = 8
    N = B * (H // HT)

    def body(in_ref, a_ref, b_ref):
        g = pl.program_id(0)

        @pl.when(g < N)
        def _():
            for y in range(HT):
                a_ref[pl.ds(y * W, W), :] = in_ref[0, :CA, y, :].T
                b_ref[pl.ds(y * W, W), :CB] = in_ref[0, CA:, y, :].T

        @pl.when(g >= N)
        def _():
            a_ref[...] = jnp.zeros((HT * W, CA), jnp.float32)
            b_ref[...] = jnp.zeros((HT * W, CA), jnp.float32)

    return pl.pallas_call(
        body,
        grid=(N + 1,),
        in_specs=[pl.BlockSpec(
            (1, C, HT, W),
            lambda g: (jnp.minimum(g // (H // HT), B - 1),
                       0, jnp.minimum(g % (H // HT), H // HT - 1), 0))],
        out_specs=[pl.BlockSpec((HT * W, CA), lambda g: (g, 0)),
                   pl.BlockSpec((HT * W, CA), lambda g: (g, 0))],
        out_shape=[jax.ShapeDtypeStruct(((N + 1) * HT * W, CA), jnp.float32),
                   jax.ShapeDtypeStruct(((N + 1) * HT * W, CA), jnp.float32)],
    )(features)


def _roi_align_sc(table_a, table_b, rois16, K, H, W):
    info = plsc.get_sparse_core_info()
    NC, NS = info.num_cores, info.num_subcores
    NW = NC * NS
    RPW = K // NW
    assert RPW * NW == K

    mesh = plsc.VectorSubcoreMesh(core_axis_name="c", subcore_axis_name="s")
    f32, i32 = jnp.float32, jnp.int32

    def body(ta_ref, tb_ref, rois_ref, out_ref, roi_v, ybase_s, hy_s, ly_s,
             w_ref, idx0, idx1, idx2, idx3, idx4, idx5, idx6,
             rowsa0, rowsa1, rowsb0, rowsb1, acc, sem0, sem1):
        idx_refs = [idx0, idx1, idx2, idx3, idx4, idx5, idx6]
        wid = lax.axis_index("s") * NC + lax.axis_index("c")
        lane = lax.iota(i32, 16)
        lanef = lane.astype(f32)
        active = lane < S

        def bc(ref, i):
            return plsc.load_gather(ref, [jnp.full((16,), i, i32)])

        def run_roi(kk, carry):
            k = wid * RPW + kk
            pltpu.sync_copy(rois_ref.at[k], roi_v)
            bf = bc(roi_v, 0)
            x1 = bc(roi_v, 1)
            y1 = bc(roi_v, 2)
            x2 = bc(roi_v, 3)
            y2 = bc(roi_v, 4)
            bw = jnp.maximum(x2 - x1, 1.0) * (1.0 / P)
            bh = jnp.maximum(y2 - y1, 1.0) * (1.0 / P)
            off = lanef * (1.0 / G) + (0.5 / G)
            xs = x1 + off * bw
            ys = y1 + off * bh
            cx = jnp.maximum(xs, 0.0)
            xli = cx.astype(i32)
            xcond = xli >= (W - 1)
            xli = jnp.where(xcond, W - 1, xli)
            lx = jnp.where(xcond, 0.0, cx - xli.astype(f32))
            hx = 1.0 - lx
            xli = jnp.where(active, xli, 0)
            lx = jnp.where(active, lx, 0.0)
            hx = jnp.where(active, hx, 0.0)
            cy = jnp.maximum(ys, 0.0)
            yli = cy.astype(i32)
            ycond = yli >= (H - 1)
            yli = jnp.where(ycond, H - 1, yli)
            ly = jnp.where(ycond, 0.0, cy - yli.astype(f32))
            hy = 1.0 - ly
            ybase = bf.astype(i32) * (H * W) + yli * W
            ybase_s[...] = ybase
            hy_s[...] = hy * 0.25
            ly_s[...] = ly * 0.25

            for iy in range(S):
                yb = bc(ybase_s, iy)
                hyb = bc(hy_s, iy)
                lyb = bc(ly_s, iy)
                ill = yb + xli
                idxj = idx_refs[iy // 2]
                base = (iy % 2) * 64
                idxj[pl.ds(base, 16)] = ill
                idxj[pl.ds(base + 16, 16)] = ill + 1
                idxj[pl.ds(base + 32, 16)] = ill + W
                idxj[pl.ds(base + 48, 16)] = ill + (W + 1)
                w_ref[pl.ds(iy * 64, 16)] = hyb * hx
                w_ref[pl.ds(iy * 64 + 16, 16)] = hyb * lx
                w_ref[pl.ds(iy * 64 + 32, 16)] = lyb * hx
                w_ref[pl.ds(iy * 64 + 48, 16)] = lyb * lx

            @plsc.parallel_loop(0, P * P, unroll=2)
            def _zero(i):
                zb = i * NCH
                for cv in range(CV):
                    acc[pl.ds(zb + cv * 16, 16)] = jnp.zeros((16,), f32)

            bufsa = [rowsa0, rowsa1]
            bufsb = [rowsb0, rowsb1]
            sems = [sem0, sem1]

            def fire(j):
                par = j % 2
                return (pltpu.async_copy(ta_ref.at[idx_refs[j]],
                                         bufsa[par], sems[par]),
                        pltpu.async_copy(tb_ref.at[idx_refs[j]],
                                         bufsb[par], sems[par]))

            cps = {0: fire(0)}
            for j in range(P):
                cps[j][0].wait()
                cps[j][1].wait()
                if j + 1 < P:
                    cps[j + 1] = fire(j + 1)
                ra = bufsa[j % 2]
                rb_buf = bufsb[j % 2]

                @plsc.parallel_loop(0, 2 * S, unroll=2)
                def _point(i, ra=ra, rb_buf=rb_buf, j=j):
                    hh = (i >= S).astype(i32)
                    ix = i - hh * S
                    wb = (2 * j + hh) * 64 + ix
                    w0 = plsc.load_gather(w_ref, [jnp.full((16,), wb, i32)])
                    w1 = plsc.load_gather(w_ref, [jnp.full((16,), wb + 16, i32)])
                    w2 = plsc.load_gather(w_ref, [jnp.full((16,), wb + 32, i32)])
                    w3 = plsc.load_gather(w_ref, [jnp.full((16,), wb + 48, i32)])
                    accoff = (j * P + ix // 2) * NCH
                    r0 = hh * 64 + ix
                    for cv in range(CV):
                        if cv < CA // 16:
                            co = pl.ds(cv * 16, 16)
                            v = (w0 * ra[r0, co] + w1 * ra[r0 + 16, co]
                                 + w2 * ra[r0 + 32, co] + w3 * ra[r0 + 48, co])
                        else:
                            co = pl.ds((cv - CA // 16) * 16, 16)
                            v = (w0 * rb_buf[r0, co] + w1 * rb_buf[r0 + 16, co]
                                 + w2 * rb_buf[r0 + 32, co] + w3 * rb_buf[r0 + 48, co])
                        plsc.addupdate(acc.at[pl.ds(accoff + cv * 16, 16)], v)
            pltpu.sync_copy(acc, out_ref.at[k])
            return carry

        lax.fori_loop(0, RPW, run_roi, 0)

    fn = pl.kernel(
        body,
        out_type=jax.ShapeDtypeStruct((K, P * P * NCH), jnp.float32),
        mesh=mesh,
        compiler_params=pltpu.CompilerParams(
            needs_layout_passes=False, use_tc_tiling_on_sc=False),
        scratch_types=(
            [pltpu.VMEM((16,), f32),
             pltpu.VMEM((16,), i32),
             pltpu.VMEM((16,), f32),
             pltpu.VMEM((16,), f32)]
            + [pltpu.VMEM((S * 64,), f32)]
            + [pltpu.VMEM((128,), i32) for _ in range(P)]
            + [pltpu.VMEM((128, CA), f32),
               pltpu.VMEM((128, CA), f32),
               pltpu.VMEM((128, CA), f32),
               pltpu.VMEM((128, CA), f32),
               pltpu.VMEM((P * P * NCH,), f32),
               pltpu.SemaphoreType.DMA,
               pltpu.SemaphoreType.DMA]),
    )
    return fn(table_a, table_b, rois16)


@jax.jit
def kernel(features, rois):
    B, C, H, W = features.shape
    K = rois.shape[0]
    table_a, table_b = _nchw_to_tables(features)
    rois16 = jnp.pad(rois, ((0, 0), (0, 11)))
    flat = _roi_align_sc(table_a, table_b, rois16, K, H, W)
    return flat.reshape(K, P * P, C).transpose(0, 2, 1).reshape(K, C, P, P)
